# baseline (speedup 1.0000x reference)
import functools

import jax
import jax.numpy as jnp
from jax import lax
from jax.experimental import pallas as pl
from jax.experimental.pallas import tpu as pltpu
from jax.experimental.pallas import tpu_sc as plsc

N = 10000
NE = 10000
E = 320000
D = 128

NC = 2
NS = 16
NW = NC * NS
T = E // NW
K = 128
T_PAD = 10240
NB = T_PAD // K
PAD_SEG = 10100
S_PAD = 10240
ROWS_PER_TILE = S_PAD // NS


def _agg_body(table, cidx, zeros, out, cidx_v, rows_v, acc_sh,
              si0, si1, si2, si3, sg0, sg1):
    c = lax.axis_index("c")
    s = lax.axis_index("s")
    wid = c * NS + s
    isems = (si0, si1, si2, si3)
    gsems = (sg0, sg1)

    pltpu.sync_copy(zeros, rows_v.at[0])
    for r in range(ROWS_PER_TILE // K):
        pltpu.sync_copy(rows_v.at[0], acc_sh.at[pl.ds(s * ROWS_PER_TILE + r * K, K)])
    plsc.subcore_barrier()

    def istart(j, b):
        pltpu.async_copy(cidx.at[wid, j], cidx_v.at[b], isems[b])

    def iwait(b):
        pltpu.make_async_copy(cidx.at[0, 0], cidx_v.at[b], isems[b]).wait()

    def gstart(j, b, ib):
        pltpu.async_copy(table.at[cidx_v.at[ib, 0]], rows_v.at[b], gsems[b])

    def gwait(b):
        pltpu.make_async_copy(table.at[cidx_v.at[0, 0]], rows_v.at[b],
                              gsems[b]).wait()

    def scat(j, b, ib):
        pltpu.sync_copy(rows_v.at[b], acc_sh.at[cidx_v.at[ib, 1]], add=True)

    for b in range(4):
        istart(b, b)

    def quad(i, carry):
        j = 4 * i
        for q in range(4):
            iwait(q)
            gstart(j + q, q & 1, q)
            gwait(q & 1)
            scat(j + q, q & 1, q)
            istart(j + 4 + q, q)
        return carry

    lax.fori_loop(0, NB // 4 - 1, quad, 0)
    j = NB - 4
    for q in range(4):
        iwait(q)
        gstart(j + q, q & 1, q)
        gwait(q & 1)
        scat(j + q, q & 1, q)

    plsc.subcore_barrier()
    for r in range(ROWS_PER_TILE // K):
        sl = pl.ds(s * ROWS_PER_TILE + r * K, K)
        pltpu.sync_copy(acc_sh.at[sl], rows_v.at[0])
        pltpu.sync_copy(rows_v.at[0], out.at[c, sl])


def _make_agg():
    mesh = plsc.VectorSubcoreMesh(core_axis_name="c", subcore_axis_name="s")
    return pl.kernel(
        _agg_body,
        out_type=jax.ShapeDtypeStruct((NC, S_PAD, D), jnp.float32),
        mesh=mesh,
        scratch_types=[
            pltpu.VMEM((4, 2, K), jnp.int32),
            pltpu.VMEM((2, K, D), jnp.float32),
            pltpu.VMEM_SHARED((S_PAD, D), jnp.float32),
            pltpu.SemaphoreType.DMA,
            pltpu.SemaphoreType.DMA,
            pltpu.SemaphoreType.DMA,
            pltpu.SemaphoreType.DMA,
            pltpu.SemaphoreType.DMA,
            pltpu.SemaphoreType.DMA,
        ],
    )


def _combine_body(relu, p_ref, cnt_ref, w_ref, b_ref, o_ref):
    ssum = p_ref[0] + p_ref[1]
    cnt = cnt_ref[0] + cnt_ref[1]
    mean = ssum / jnp.maximum(cnt, 1.0)
    y = jnp.dot(mean, w_ref[...], preferred_element_type=jnp.float32)
    y = y[:NE] + b_ref[...][None, :]
    if relu:
        y = jnp.maximum(y, 0.0)
    o_ref[...] = y


def _combine(partials, cnts, w, b, relu):
    body = functools.partial(_combine_body, relu)
    return pl.pallas_call(
        body,
        out_shape=jax.ShapeDtypeStruct((NE, D), jnp.float32),
    )(partials, cnts, w, b)


def _stack_idx(g, s_):
    g2 = jnp.pad(g.reshape(NW, T), ((0, 0), (0, T_PAD - T)))
    s2 = jnp.pad(s_.reshape(NW, T), ((0, 0), (0, T_PAD - T)),
                 constant_values=PAD_SEG)
    return jnp.stack([g2.reshape(NW, NB, K), s2.reshape(NW, NB, K)], axis=2)


def kernel(x, ei, W1_e, b1_e, W1_n, b1_n, W2_e, b2_e, W2_n, b2_n):
    ci_ne = _stack_idx(ei[0], ei[1])
    ci_en = _stack_idx(ei[1], ei[0])
    zeros_b = jnp.zeros((K, D), jnp.float32)
    ones_t = jnp.ones((N, D), jnp.float32)

    agg = _make_agg()
    cnt_e = agg(ones_t, ci_ne, zeros_b)
    cnt_n = agg(ones_t, ci_en, zeros_b)

    h = x
    for (We, be, Wn, bn) in ((W1_e, b1_e, W1_n, b1_n), (W2_e, b2_e, W2_n, b2_n)):
        ep = agg(h, ci_ne, zeros_b)
        ef = _combine(ep, cnt_e, We, be, relu=False)
        np_ = agg(ef, ci_en, zeros_b)
        h = _combine(np_, cnt_n, Wn, bn, relu=True)
    return h


# speedup vs baseline: 1.1580x; 1.1580x over previous
import functools

import jax
import jax.numpy as jnp
from jax import lax
from jax.experimental import pallas as pl
from jax.experimental.pallas import tpu as pltpu
from jax.experimental.pallas import tpu_sc as plsc

N = 10000
NE = 10000
E = 320000
D = 128

NC = 2
NS = 16
NW = NC * NS
T = E // NW
K = 128
T_PAD = 10240
NB = T_PAD // K
PAD_SEG = 10100
S_PAD = 10240
ROWS_PER_TILE = S_PAD // NS


def _agg_body(table, cidx, zeros, out, cidx_v, rows_v, acc_sh,
              si0, si1, si2, si3, sg0, sg1):
    c = lax.axis_index("c")
    s = lax.axis_index("s")
    wid = c * NS + s
    isems = (si0, si1, si2, si3)
    gsems = (sg0, sg1)

    pltpu.sync_copy(zeros, rows_v.at[0])
    for r in range(ROWS_PER_TILE // K):
        pltpu.sync_copy(rows_v.at[0], acc_sh.at[pl.ds(s * ROWS_PER_TILE + r * K, K)])
    plsc.subcore_barrier()

    def istart(j, b):
        pltpu.async_copy(cidx.at[wid, j], cidx_v.at[b], isems[b])

    def iwait(b):
        pltpu.make_async_copy(cidx.at[0, 0], cidx_v.at[b], isems[b]).wait()

    def gstart(j, b, ib):
        pltpu.async_copy(table.at[cidx_v.at[ib, 0]], rows_v.at[b], gsems[b])

    def gwait(b):
        pltpu.make_async_copy(table.at[cidx_v.at[0, 0]], rows_v.at[b],
                              gsems[b]).wait()

    def scat(j, b, ib):
        pltpu.sync_copy(rows_v.at[b], acc_sh.at[cidx_v.at[ib, 1]], add=True)

    for b in range(4):
        istart(b, b)
    iwait(0)
    gstart(0, 0, 0)

    def quad(i, carry):
        j = 4 * i
        iwait(1)
        gstart(j + 1, 1, 1)
        gwait(0)
        scat(j, 0, 0)
        istart(j + 4, 0)
        iwait(2)
        gstart(j + 2, 0, 2)
        gwait(1)
        scat(j + 1, 1, 1)
        istart(j + 5, 1)
        iwait(3)
        gstart(j + 3, 1, 3)
        gwait(0)
        scat(j + 2, 0, 2)
        istart(j + 6, 2)
        iwait(0)
        gstart(j + 4, 0, 0)
        gwait(1)
        scat(j + 3, 1, 3)
        istart(j + 7, 3)
        return carry

    lax.fori_loop(0, NB // 4 - 1, quad, 0)
    j = NB - 4
    iwait(1)
    gstart(j + 1, 1, 1)
    gwait(0)
    scat(j, 0, 0)
    iwait(2)
    gstart(j + 2, 0, 2)
    gwait(1)
    scat(j + 1, 1, 1)
    iwait(3)
    gstart(j + 3, 1, 3)
    gwait(0)
    scat(j + 2, 0, 2)
    gwait(1)
    scat(j + 3, 1, 3)

    plsc.subcore_barrier()
    for r in range(ROWS_PER_TILE // K):
        sl = pl.ds(s * ROWS_PER_TILE + r * K, K)
        pltpu.sync_copy(acc_sh.at[sl], rows_v.at[0])
        pltpu.sync_copy(rows_v.at[0], out.at[c, sl])


def _make_agg():
    mesh = plsc.VectorSubcoreMesh(core_axis_name="c", subcore_axis_name="s")
    return pl.kernel(
        _agg_body,
        out_type=jax.ShapeDtypeStruct((NC, S_PAD, D), jnp.float32),
        mesh=mesh,
        scratch_types=[
            pltpu.VMEM((4, 2, K), jnp.int32),
            pltpu.VMEM((2, K, D), jnp.float32),
            pltpu.VMEM_SHARED((S_PAD, D), jnp.float32),
            pltpu.SemaphoreType.DMA,
            pltpu.SemaphoreType.DMA,
            pltpu.SemaphoreType.DMA,
            pltpu.SemaphoreType.DMA,
            pltpu.SemaphoreType.DMA,
            pltpu.SemaphoreType.DMA,
        ],
    )


def _combine_body(relu, p_ref, cnt_ref, w_ref, b_ref, o_ref):
    ssum = p_ref[0] + p_ref[1]
    cnt = cnt_ref[0] + cnt_ref[1]
    mean = ssum / jnp.maximum(cnt, 1.0)
    y = jnp.dot(mean, w_ref[...], preferred_element_type=jnp.float32)
    y = y[:NE] + b_ref[...][None, :]
    if relu:
        y = jnp.maximum(y, 0.0)
    o_ref[...] = y


def _combine(partials, cnts, w, b, relu):
    body = functools.partial(_combine_body, relu)
    return pl.pallas_call(
        body,
        out_shape=jax.ShapeDtypeStruct((NE, D), jnp.float32),
    )(partials, cnts, w, b)


def _stack_idx(g, s_):
    npad = T_PAD - T
    g2 = jnp.pad(g.reshape(NW, T), ((0, 0), (0, npad)))
    pads = (jnp.arange(NW, dtype=jnp.int32)[:, None]
            + jnp.arange(npad, dtype=jnp.int32)[None, :]) % (S_PAD - NE) + NE
    s2 = jnp.concatenate([s_.reshape(NW, T), pads], axis=1)
    return jnp.stack([g2.reshape(NW, NB, K), s2.reshape(NW, NB, K)], axis=2)


def kernel(x, ei, W1_e, b1_e, W1_n, b1_n, W2_e, b2_e, W2_n, b2_n):
    ci_ne = _stack_idx(ei[0], ei[1])
    ci_en = _stack_idx(ei[1], ei[0])
    zeros_b = jnp.zeros((K, D), jnp.float32)
    ones_t = jnp.ones((N, D), jnp.float32)

    agg = _make_agg()
    cnt_e = agg(ones_t, ci_ne, zeros_b)
    cnt_n = agg(ones_t, ci_en, zeros_b)

    h = x
    for (We, be, Wn, bn) in ((W1_e, b1_e, W1_n, b1_n), (W2_e, b2_e, W2_n, b2_n)):
        ep = agg(h, ci_ne, zeros_b)
        ef = _combine(ep, cnt_e, We, be, relu=False)
        np_ = agg(ef, ci_en, zeros_b)
        h = _combine(np_, cnt_n, Wn, bn, relu=True)
    return h
